# baseline (speedup 1.0000x reference)
import jax
import jax.numpy as jnp
from jax.experimental import pallas as pl

_LEVELS = (8, 8, 8, 5, 5, 5)
_BASIS = (1.0, 8.0, 64.0, 512.0, 2560.0, 12800.0)
_D = len(_LEVELS)
_B, _S = 32, 1024
_GRID = 4
_SB = _S // _GRID
_RNE = 1.5 * 2.0**23


def _fsq_body(z_ref, q_ref, idx_ref, li_ref):
    acc = jnp.zeros((_B, _SB), jnp.float32)
    for j in range(_D):
        x = z_ref[j]
        act = (jnp.tanh(x) + 1.0) * 0.5
        y = act * jnp.float32(_LEVELS[j] - 1)
        lif = (y + _RNE) - _RNE
        q_ref[j] = (lif / jnp.float32(_LEVELS[j] - 1)) * 2.0 - 1.0
        li_ref[j] = lif.astype(jnp.int32)
        acc = acc + lif * jnp.float32(_BASIS[j])
    idx_ref[...] = acc.astype(jnp.int32)


_plane_spec = pl.BlockSpec((_D, _B, _SB), lambda g: (0, 0, g))

_fsq_tc = pl.pallas_call(
    _fsq_body,
    grid=(_GRID,),
    in_specs=[_plane_spec],
    out_specs=[
        _plane_spec,
        pl.BlockSpec((_B, _SB), lambda g: (0, g)),
        _plane_spec,
    ],
    out_shape=[
        jax.ShapeDtypeStruct((_D, _B, _S), jnp.float32),
        jax.ShapeDtypeStruct((_B, _S), jnp.int32),
        jax.ShapeDtypeStruct((_D, _B, _S), jnp.int32),
    ],
)


def kernel(z):
    q, idx, li = _fsq_tc(z.transpose(2, 0, 1))
    return q.transpose(1, 2, 0), idx, li.transpose(1, 2, 0)


# speedup vs baseline: 1.0158x; 1.0158x over previous
import jax
import jax.numpy as jnp
from jax.experimental import pallas as pl

_LEVELS = (8, 8, 8, 5, 5, 5)
_BASIS = (1.0, 8.0, 64.0, 512.0, 2560.0, 12800.0)
_D = len(_LEVELS)
_B, _S = 32, 1024
_GRID = 4
_BB = _B // _GRID
_RNE = 1.5 * 2.0**23


def _fsq_body(z_ref, q_ref, idx_ref, li_ref):
    acc = jnp.zeros((_BB, _S), jnp.float32)
    for j in range(_D):
        x = z_ref[j]
        act = (jnp.tanh(x) + 1.0) * 0.5
        y = act * jnp.float32(_LEVELS[j] - 1)
        lif = (y + _RNE) - _RNE
        q_ref[j] = (lif / jnp.float32(_LEVELS[j] - 1)) * 2.0 - 1.0
        li_ref[j] = lif.astype(jnp.int32)
        acc = acc + lif * jnp.float32(_BASIS[j])
    idx_ref[...] = acc.astype(jnp.int32)


_plane_spec = pl.BlockSpec((_D, _BB, _S), lambda g: (0, g, 0))

_fsq_tc = pl.pallas_call(
    _fsq_body,
    grid=(_GRID,),
    in_specs=[_plane_spec],
    out_specs=[
        _plane_spec,
        pl.BlockSpec((_BB, _S), lambda g: (g, 0)),
        _plane_spec,
    ],
    out_shape=[
        jax.ShapeDtypeStruct((_D, _B, _S), jnp.float32),
        jax.ShapeDtypeStruct((_B, _S), jnp.int32),
        jax.ShapeDtypeStruct((_D, _B, _S), jnp.int32),
    ],
)


def kernel(z):
    q, idx, li = _fsq_tc(z.transpose(2, 0, 1))
    return q.transpose(1, 2, 0), idx, li.transpose(1, 2, 0)


# speedup vs baseline: 1.1514x; 1.1334x over previous
import jax
import jax.numpy as jnp
from jax.experimental import pallas as pl
from jax.experimental.pallas import tpu as pltpu

_LEVELS = (8, 8, 8, 5, 5, 5)
_BASIS = (1.0, 8.0, 64.0, 512.0, 2560.0, 12800.0)
_D = len(_LEVELS)
_B, _S = 32, 1024
_CH = 4
_BB = _B // _CH
_RNE = 1.5 * 2.0**23


def _fsq_body(z_hbm, q_hbm, idx_hbm, li_hbm, zv, qv, iv, liv, insem, outsem):
    def in_copy(g):
        return pltpu.make_async_copy(
            z_hbm.at[:, pl.ds(g * _BB, _BB), :], zv.at[g % 2], insem.at[g % 2])

    def out_copies(g):
        s = g % 2
        return [
            pltpu.make_async_copy(qv.at[s], q_hbm.at[:, pl.ds(g * _BB, _BB), :], outsem.at[s]),
            pltpu.make_async_copy(liv.at[s], li_hbm.at[:, pl.ds(g * _BB, _BB), :], outsem.at[s]),
            pltpu.make_async_copy(iv.at[s], idx_hbm.at[pl.ds(g * _BB, _BB), :], outsem.at[s]),
        ]

    in_copy(0).start()
    pending = {}
    for g in range(_CH):
        if g + 1 < _CH:
            in_copy(g + 1).start()
        in_copy(g).wait()
        if g >= 2:
            for c in pending.pop(g - 2):
                c.wait()
        s = g % 2
        acc = jnp.zeros((_BB, _S), jnp.float32)
        for j in range(_D):
            x = zv[s, j]
            act = (jnp.tanh(x) + 1.0) * 0.5
            y = act * jnp.float32(_LEVELS[j] - 1)
            lif = (y + _RNE) - _RNE
            qv[s, j] = (lif / jnp.float32(_LEVELS[j] - 1)) * 2.0 - 1.0
            liv[s, j] = lif.astype(jnp.int32)
            acc = acc + lif * jnp.float32(_BASIS[j])
        iv[s] = acc.astype(jnp.int32)
        cs = out_copies(g)
        for c in cs:
            c.start()
        pending[g] = cs
    for g in sorted(pending):
        for c in pending[g]:
            c.wait()


_fsq_tc = pl.pallas_call(
    _fsq_body,
    in_specs=[pl.BlockSpec(memory_space=pl.ANY)],
    out_specs=[
        pl.BlockSpec(memory_space=pl.ANY),
        pl.BlockSpec(memory_space=pl.ANY),
        pl.BlockSpec(memory_space=pl.ANY),
    ],
    out_shape=[
        jax.ShapeDtypeStruct((_D, _B, _S), jnp.float32),
        jax.ShapeDtypeStruct((_B, _S), jnp.int32),
        jax.ShapeDtypeStruct((_D, _B, _S), jnp.int32),
    ],
    scratch_shapes=[
        pltpu.VMEM((2, _D, _BB, _S), jnp.float32),
        pltpu.VMEM((2, _D, _BB, _S), jnp.float32),
        pltpu.VMEM((2, _BB, _S), jnp.int32),
        pltpu.VMEM((2, _D, _BB, _S), jnp.int32),
        pltpu.SemaphoreType.DMA((2,)),
        pltpu.SemaphoreType.DMA((2,)),
    ],
)


def kernel(z):
    q, idx, li = _fsq_tc(z.transpose(2, 0, 1))
    return q.transpose(1, 2, 0), idx, li.transpose(1, 2, 0)
